# baseline (speedup 1.0000x reference)
import jax
import jax.numpy as jnp
from jax.experimental import pallas as pl
from jax.experimental.pallas import tpu as pltpu

_B, _N, _D = 16, 2048, 3
_DP = 8


def _chamfer_body(p_ref, gt_ref, out_ref):
    b = pl.program_id(0)
    p = p_ref[0]
    g = gt_ref[0]

    p_sq = jnp.sum(p * p, axis=1, keepdims=True)
    g_sq = jnp.sum(g * g, axis=0, keepdims=True)

    cross = jnp.dot(p, g, preferred_element_type=jnp.float32)

    d = p_sq + g_sq - 2.0 * cross

    s = jnp.sum(jnp.min(d, axis=1)) + jnp.sum(jnp.min(d, axis=0))

    @pl.when(b == 0)
    def _():
        out_ref[0, 0] = 0.0

    out_ref[0, 0] += s

    @pl.when(b == _B - 1)
    def _():
        out_ref[0, 0] = out_ref[0, 0] * (1.0 / (_B * _N))


def kernel(prediction, gt):
    p_pad = jnp.pad(prediction, ((0, 0), (0, 0), (0, _DP - _D)))
    g_t = jnp.pad(jnp.swapaxes(gt, 1, 2), ((0, 0), (0, _DP - _D), (0, 0)))

    out = pl.pallas_call(
        _chamfer_body,
        grid=(_B,),
        in_specs=[
            pl.BlockSpec((1, _N, _DP), lambda b: (b, 0, 0)),
            pl.BlockSpec((1, _DP, _N), lambda b: (b, 0, 0)),
        ],
        out_specs=pl.BlockSpec(memory_space=pltpu.SMEM),
        out_shape=jax.ShapeDtypeStruct((1, 1), jnp.float32),
        compiler_params=pltpu.CompilerParams(
            dimension_semantics=("arbitrary",),
        ),
    )(p_pad, g_t)
    return out[0, 0]


# speedup vs baseline: 1.1716x; 1.1716x over previous
import jax
import jax.numpy as jnp
from jax.experimental import pallas as pl
from jax.experimental.pallas import tpu as pltpu

_B, _N, _D = 16, 2048, 3
_DP = 8


def _chamfer_body(p_ref, gt_ref, out_ref):
    b = pl.program_id(0)
    p = p_ref[0]
    g = gt_ref[0]

    p_sq = jnp.sum(p * p, axis=1, keepdims=True)
    g_sq = jnp.sum(g * g, axis=0, keepdims=True)

    n = p.shape[0]
    a = jnp.concatenate(
        [-2.0 * p[:, :_D], p_sq, jnp.ones((n, 1), jnp.float32),
         jnp.zeros((n, _DP - _D - 2), jnp.float32)], axis=1)
    bg = jnp.concatenate(
        [g[:_D, :], jnp.ones((1, n), jnp.float32), g_sq,
         jnp.zeros((_DP - _D - 2, n), jnp.float32)], axis=0)

    d = jnp.dot(a, bg, preferred_element_type=jnp.float32)

    s = jnp.sum(jnp.min(d, axis=1)) + jnp.sum(jnp.min(d, axis=0))

    @pl.when(b == 0)
    def _():
        out_ref[0, 0] = 0.0

    out_ref[0, 0] += s

    @pl.when(b == _B - 1)
    def _():
        out_ref[0, 0] = out_ref[0, 0] * (1.0 / (_B * _N))


def kernel(prediction, gt):
    p_pad = jnp.pad(prediction, ((0, 0), (0, 0), (0, _DP - _D)))
    g_t = jnp.pad(jnp.swapaxes(gt, 1, 2), ((0, 0), (0, _DP - _D), (0, 0)))

    out = pl.pallas_call(
        _chamfer_body,
        grid=(_B,),
        in_specs=[
            pl.BlockSpec((1, _N, _DP), lambda b: (b, 0, 0)),
            pl.BlockSpec((1, _DP, _N), lambda b: (b, 0, 0)),
        ],
        out_specs=pl.BlockSpec(memory_space=pltpu.SMEM),
        out_shape=jax.ShapeDtypeStruct((1, 1), jnp.float32),
        compiler_params=pltpu.CompilerParams(
            dimension_semantics=("arbitrary",),
        ),
    )(p_pad, g_t)
    return out[0, 0]
